# baseline (speedup 1.0000x reference)
import functools

import jax
import jax.numpy as jnp
from jax import lax
from jax.experimental import pallas as pl
from jax.experimental.pallas import tpu as pltpu
from jax.experimental.pallas import tpu_sc as plsc

EMB_DIM = 64
EMB_NUM = 12
NPAIR = EMB_NUM ** 2
NQUAD = EMB_NUM ** 4
PD = 2 * EMB_DIM
QD = 4 * EMB_DIM

NC = 2
NS = 16
NW = NC * NS

QCH = 64
PCH = 128
NBUF = 2


def _prep_body(pe_ref, gamma_ref, beta_ref, w_ref, b_ref,
               xa_ref, xb_ref, xc_ref, xd_ref, tab4_ref, tab2_ref, qidx_ref,
               p0_ref, p1_ref):
    n = EMB_NUM
    row = lax.broadcasted_iota(jnp.int32, (n, n), 0)
    col = lax.broadcasted_iota(jnp.int32, (n, n), 1)
    h = jnp.where(row == col, jnp.float32(1.0), jnp.float32(0.0)) + jnp.float32(0.05)
    mean = jnp.mean(h, axis=1, keepdims=True)
    var = jnp.mean((h - mean) ** 2, axis=1, keepdims=True)
    hn = (h - mean) / jnp.sqrt(var + jnp.float32(1e-5))
    hn = hn * gamma_ref[...] + beta_ref[...]
    t = lax.dot_general(hn, w_ref[...], (((1,), (1,)), ((), ())),
                        preferred_element_type=jnp.float32)
    t = jnp.maximum(t + b_ref[...], jnp.float32(0.0)) + pe_ref[...]

    q_iota = lax.broadcasted_iota(jnp.int32, (NQUAD, n), 0)
    qc_iota = lax.broadcasted_iota(jnp.int32, (NQUAD, n), 1)
    for k, div in enumerate((n ** 3, n ** 2, n, 1)):
        sel = ((q_iota // div) % n == qc_iota).astype(jnp.float32)
        tab4_ref[:, k * EMB_DIM:(k + 1) * EMB_DIM] = lax.dot_general(
            sel, t, (((1,), (0,)), ((), ())), preferred_element_type=jnp.float32)

    p_iota = lax.broadcasted_iota(jnp.int32, (NPAIR, n), 0)
    pc_iota = lax.broadcasted_iota(jnp.int32, (NPAIR, n), 1)
    for k, div in enumerate((n, 1)):
        sel = ((p_iota // div) % n == pc_iota).astype(jnp.float32)
        tab2_ref[:, k * EMB_DIM:(k + 1) * EMB_DIM] = lax.dot_general(
            sel, t, (((1,), (0,)), ((), ())), preferred_element_type=jnp.float32)

    qidx_ref[...] = ((xa_ref[...] * n + xb_ref[...]) * n + xc_ref[...]) * n \
        + xd_ref[...]
    p0_ref[...] = xa_ref[...] * n + xb_ref[...]
    p1_ref[...] = xc_ref[...] * n + xd_ref[...]


def _prep(pe, gamma, beta, W, b, xa, xb, xc, xd):
    nchunks = xa.shape[0]
    return pl.pallas_call(
        _prep_body,
        out_shape=[
            jax.ShapeDtypeStruct((NQUAD, QD), jnp.float32),
            jax.ShapeDtypeStruct((NPAIR, PD), jnp.float32),
            jax.ShapeDtypeStruct((nchunks, QCH), jnp.int32),
            jax.ShapeDtypeStruct((nchunks, QCH), jnp.int32),
            jax.ShapeDtypeStruct((nchunks, QCH), jnp.int32),
        ],
    )(pe, gamma.reshape(1, EMB_NUM), beta.reshape(1, EMB_NUM),
      W, b.reshape(1, EMB_DIM), xa, xb, xc, xd)


def _make_gather(total_quads):
    per_w = total_quads // NW
    half = per_w // 2
    n_steps = half // QCH
    assert n_steps % NBUF == 0
    n_groups = n_steps // NBUF
    mesh = plsc.VectorSubcoreMesh(core_axis_name="c", subcore_axis_name="s")

    @functools.partial(
        pl.kernel,
        mesh=mesh,
        out_type=jax.ShapeDtypeStruct((total_quads, QD), jnp.float32),
        scratch_types=[
            pltpu.VMEM_SHARED((NPAIR, PD), jnp.float32),
            pltpu.VMEM((half,), jnp.int32),
            pltpu.VMEM((2 * half,), jnp.int32),
            [pltpu.VMEM((QCH, QD), jnp.float32) for _ in range(NBUF)],
            [pltpu.VMEM((PCH, PD), jnp.float32) for _ in range(NBUF)],
            [pltpu.SemaphoreType.DMA for _ in range(NBUF)],
            [pltpu.SemaphoreType.DMA for _ in range(NBUF)],
            [pltpu.SemaphoreType.DMA for _ in range(NBUF)],
            [pltpu.SemaphoreType.DMA for _ in range(NBUF)],
        ],
    )
    def gather_kernel(tab4_hbm, tab2_hbm, qidx_hbm, pidx_hbm, out_hbm,
                      tab2_v, qidx_v, pidx_v, rowsq, rowsp,
                      gq, gp, sq, sp):
        sid = lax.axis_index("s")
        wid = sid * NC + lax.axis_index("c")
        qbase = wid * per_w
        pbase = qbase + half

        @pl.when(sid == 0)
        def _stage_table():
            pltpu.sync_copy(tab2_hbm, tab2_v)

        pltpu.sync_copy(qidx_hbm.at[pl.ds(qbase, half)], qidx_v)
        pltpu.sync_copy(pidx_hbm.at[pl.ds(2 * pbase, 2 * half)], pidx_v)
        plsc.subcore_barrier()

        def group(g, carry):
            for bf in range(NBUF):
                i = g * NBUF + bf
                qoff = i * QCH

                @pl.when(g > 0)
                def _wait_prev_stores():
                    pltpu.make_async_copy(
                        rowsq[bf], out_hbm.at[pl.ds(qbase + qoff, QCH)],
                        sq[bf]).wait()
                    pltpu.make_async_copy(
                        rowsp[bf],
                        out_hbm.at[pl.ds(0, PCH), pl.ds(0, PD)],
                        sp[bf]).wait()

                pltpu.async_copy(
                    tab4_hbm.at[qidx_v.at[pl.ds(qoff, QCH)]],
                    rowsq[bf], gq[bf])
                pltpu.async_copy(
                    tab2_v.at[pidx_v.at[pl.ds(2 * qoff, PCH)]],
                    rowsp[bf], gp[bf])
            for bf in range(NBUF):
                i = g * NBUF + bf
                qoff = i * QCH
                pltpu.make_async_copy(
                    tab4_hbm.at[qidx_v.at[pl.ds(qoff, QCH)]],
                    rowsq[bf], gq[bf]).wait()
                pltpu.async_copy(rowsq[bf],
                                 out_hbm.at[pl.ds(qbase + qoff, QCH)], sq[bf])
                pltpu.make_async_copy(
                    tab2_v.at[pidx_v.at[pl.ds(2 * qoff, PCH)]],
                    rowsp[bf], gp[bf]).wait()
                pltpu.async_copy(
                    rowsp[bf].at[pl.ds(0, QCH)],
                    out_hbm.at[pl.ds(pbase + qoff, QCH), pl.ds(0, PD)],
                    sp[bf])
                pltpu.async_copy(
                    rowsp[bf].at[pl.ds(QCH, QCH)],
                    out_hbm.at[pl.ds(pbase + qoff, QCH), pl.ds(PD, PD)],
                    sp[bf])
            return carry

        lax.fori_loop(0, n_groups, group, 0)
        for bf in range(NBUF):
            pltpu.make_async_copy(
                rowsq[bf], out_hbm.at[pl.ds(qbase, QCH)], sq[bf]).wait()
            pltpu.make_async_copy(
                rowsp[bf],
                out_hbm.at[pl.ds(0, PCH), pl.ds(0, PD)], sp[bf]).wait()

    return gather_kernel


def kernel(x, pe, gamma, beta, W, b):
    Bb, Ll = x.shape
    total_quads = (Bb * Ll) // 4
    xq = x.reshape(total_quads, 4).astype(jnp.int32)
    parts = [xq[:, k].reshape(total_quads // QCH, QCH) for k in range(4)]
    tab4, tab2, qidx, p0, p1 = _prep(pe, gamma, beta, W, b, *parts)
    pidx = jnp.stack([p0, p1], axis=1).reshape(2 * total_quads)
    out = _make_gather(total_quads)(
        tab4, tab2, qidx.reshape(total_quads), pidx)
    return out.reshape(Bb, Ll, EMB_DIM)


# speedup vs baseline: 1.3498x; 1.3498x over previous
import functools

import jax
import jax.numpy as jnp
from jax import lax
from jax.experimental import pallas as pl
from jax.experimental.pallas import tpu as pltpu
from jax.experimental.pallas import tpu_sc as plsc

EMB_DIM = 64
EMB_NUM = 12
NQUAD = EMB_NUM ** 4
QD = 4 * EMB_DIM

NC = 2
NS = 16
NW = NC * NS

CHUNK = 128
NBUF = 2


def _prep_body(pe_ref, gamma_ref, beta_ref, w_ref, b_ref, x_ref,
               tab4_ref, qidx_ref):
    n = EMB_NUM
    row = lax.broadcasted_iota(jnp.int32, (n, n), 0)
    col = lax.broadcasted_iota(jnp.int32, (n, n), 1)
    h = jnp.where(row == col, jnp.float32(1.0), jnp.float32(0.0)) + jnp.float32(0.05)
    mean = jnp.mean(h, axis=1, keepdims=True)
    var = jnp.mean((h - mean) ** 2, axis=1, keepdims=True)
    hn = (h - mean) / jnp.sqrt(var + jnp.float32(1e-5))
    hn = hn * gamma_ref[...] + beta_ref[...]
    t = lax.dot_general(hn, w_ref[...], (((1,), (1,)), ((), ())),
                        preferred_element_type=jnp.float32)
    t = jnp.maximum(t + b_ref[...], jnp.float32(0.0)) + pe_ref[...]

    q_iota = lax.broadcasted_iota(jnp.int32, (NQUAD, n), 0)
    qc_iota = lax.broadcasted_iota(jnp.int32, (NQUAD, n), 1)
    for k, div in enumerate((n ** 3, n ** 2, n, 1)):
        sel = ((q_iota // div) % n == qc_iota).astype(jnp.float32)
        tab4_ref[:, k * EMB_DIM:(k + 1) * EMB_DIM] = lax.dot_general(
            sel, t, (((1,), (0,)), ((), ())), preferred_element_type=jnp.float32)

    seq_len = x_ref.shape[1]
    l_iota = lax.broadcasted_iota(jnp.int32, (seq_len, seq_len // 4), 0)
    g_iota = lax.broadcasted_iota(jnp.int32, (seq_len, seq_len // 4), 1)
    m = l_iota % 4
    pw = jnp.where(m == 0, jnp.float32(n ** 3),
                   jnp.where(m == 1, jnp.float32(n ** 2),
                             jnp.where(m == 2, jnp.float32(n), jnp.float32(1.0))))
    sq = jnp.where(l_iota // 4 == g_iota, pw, jnp.float32(0.0))
    qf = lax.dot_general(x_ref[...].astype(jnp.float32), sq,
                         (((1,), (0,)), ((), ())),
                         preferred_element_type=jnp.float32)
    qidx_ref[...] = qf.astype(jnp.int32)


def _prep(pe, gamma, beta, W, b, x):
    return pl.pallas_call(
        _prep_body,
        out_shape=[
            jax.ShapeDtypeStruct((NQUAD, QD), jnp.float32),
            jax.ShapeDtypeStruct((x.shape[0], x.shape[1] // 4), jnp.int32),
        ],
    )(pe, gamma.reshape(1, EMB_NUM), beta.reshape(1, EMB_NUM),
      W, b.reshape(1, EMB_DIM), x)


def _make_gather(total_quads):
    assert total_quads % (NW * CHUNK * NBUF) == 0
    per_w = total_quads // NW
    n_groups = per_w // (CHUNK * NBUF)
    mesh = plsc.VectorSubcoreMesh(core_axis_name="c", subcore_axis_name="s")

    @functools.partial(
        pl.kernel,
        mesh=mesh,
        out_type=jax.ShapeDtypeStruct((total_quads, QD), jnp.float32),
        scratch_types=[
            pltpu.VMEM((per_w,), jnp.int32),
            [pltpu.VMEM((CHUNK, QD), jnp.float32) for _ in range(NBUF)],
            [pltpu.SemaphoreType.DMA for _ in range(NBUF)],
            [pltpu.SemaphoreType.DMA for _ in range(NBUF)],
        ],
    )
    def gather_kernel(table_hbm, idx_hbm, out_hbm, idx_v, rows, gsems, ssems):
        sid = lax.axis_index("s")
        wid = sid * NC + lax.axis_index("c")
        base = wid * per_w
        pltpu.sync_copy(idx_hbm.at[pl.ds(base, per_w)], idx_v)

        def group(g, carry):
            for bf in range(NBUF):
                off = (g * NBUF + bf) * CHUNK

                @pl.when(g > 0)
                def _wait_prev_store():
                    pltpu.make_async_copy(
                        rows[bf], out_hbm.at[pl.ds(base + off, CHUNK)],
                        ssems[bf]).wait()

                pltpu.async_copy(
                    table_hbm.at[idx_v.at[pl.ds(off, CHUNK)]],
                    rows[bf], gsems[bf])
            for bf in range(NBUF):
                off = (g * NBUF + bf) * CHUNK
                pltpu.make_async_copy(
                    table_hbm.at[idx_v.at[pl.ds(off, CHUNK)]],
                    rows[bf], gsems[bf]).wait()
                pltpu.async_copy(rows[bf], out_hbm.at[pl.ds(base + off, CHUNK)],
                                 ssems[bf])
            return carry

        lax.fori_loop(0, n_groups, group, 0)
        for bf in range(NBUF):
            pltpu.make_async_copy(
                rows[bf], out_hbm.at[pl.ds(base, CHUNK)], ssems[bf]).wait()

    return gather_kernel


def kernel(x, pe, gamma, beta, W, b):
    Bb, Ll = x.shape
    total_quads = (Bb * Ll) // 4
    table4, qidx = _prep(pe, gamma, beta, W, b, x.astype(jnp.int32))
    out = _make_gather(total_quads)(table4, qidx.reshape(total_quads))
    return out.reshape(Bb, Ll, EMB_DIM)
